# baseline (speedup 1.0000x reference)
import jax
import jax.numpy as jnp
from jax import lax
from jax.experimental import pallas as pl
from jax.experimental.pallas import tpu as pltpu
from jax.experimental.pallas import tpu_sc as plsc

_NC, _NS, _L = 2, 16, 16
_NW = _NC * _NS


def _pack_body(a0_ref, a1_ref, a2_ref, a3_ref, out_ref):
    E = a0_ref.shape[1]
    eh = E // 2

    def rnd(x):
        b = lax.bitcast_convert_type(x, jnp.int32)
        odd = lax.bitwise_and(lax.shift_right_logical(b, 16), jnp.int32(1))
        return lax.shift_right_logical(b + jnp.int32(0x7FFF) + odd, 16)

    words = []
    for ref in (a0_ref, a1_ref, a2_ref, a3_ref):
        r = rnd(ref[...])
        words.append(lax.bitwise_or(
            r[:, :eh], lax.shift_left(r[:, eh:], jnp.int32(16))))
    out_ref[...] = jnp.concatenate(words, axis=1)


def _make_pack(V, E, BV):
    return pl.pallas_call(
        _pack_body,
        grid=(V // BV,),
        in_specs=[pl.BlockSpec((BV, E), lambda i: (i, 0))] * 4,
        out_specs=pl.BlockSpec((BV, 2 * E), lambda i: (i, 0)),
        out_shape=jax.ShapeDtypeStruct((V, 2 * E), jnp.int32),
    )


def _make_segment_sums(n_seg, S, D, CS):
    mesh = plsc.VectorSubcoreMesh(
        core_axis_name="c", subcore_axis_name="s",
        num_cores=_NC, num_subcores=_NS)
    segs_per_w = n_seg // _NW
    nchunk = segs_per_w // CS
    nb = D // (2 * _L)

    def body(ctx_hbm, tab, out, idx_v, rows_v, acc_v, gsem, osem):
        wid = lax.axis_index("s") * _NC + lax.axis_index("c")
        seg0 = wid * segs_per_w

        def start_gather(h, slot):
            idx = idx_v.at[pl.ds(h * CS * S, CS * S)]
            pltpu.async_copy(tab.at[idx], rows_v.at[slot], gsem.at[slot])

        def gather_done(slot):
            idx = idx_v.at[pl.ds(0, CS * S)]
            pltpu.make_async_copy(tab.at[idx], rows_v.at[slot],
                                  gsem.at[slot]).wait()

        def out_write(h, slot):
            pltpu.async_copy(acc_v.at[slot], out.at[pl.ds(seg0 + h * CS, CS)],
                             osem.at[slot])

        def out_done(h, slot):
            pltpu.make_async_copy(acc_v.at[slot],
                                  out.at[pl.ds(seg0 + h * CS, CS)],
                                  osem.at[slot]).wait()

        pltpu.sync_copy(ctx_hbm.at[pl.ds(seg0 * S, segs_per_w * S)], idx_v)
        start_gather(0, 0)

        def chunk_body(h, _):
            slot = lax.rem(h, 2)

            @pl.when(h + 1 < nchunk)
            def _():
                start_gather(h + 1, 1 - slot)

            gather_done(slot)

            @pl.when(h >= 2)
            def _():
                out_done(h - 2, slot)

            def unpack(x):
                lo = plsc.bitcast(lax.shift_left(x, jnp.int32(16)),
                                  jnp.float32)
                hi = plsc.bitcast(lax.bitwise_and(x, jnp.int32(-65536)),
                                  jnp.float32)
                return lo, hi

            def seg_body(si, _):
                r = si * S
                acc_lo = [None] * nb
                acc_hi = [None] * nb
                for c in range(nb):
                    acc_lo[c], acc_hi[c] = unpack(
                        rows_v[slot, r, pl.ds(c * _L, _L)])
                for s in range(1, S):
                    for c in range(nb):
                        lo, hi = unpack(
                            rows_v[slot, r + s, pl.ds(c * _L, _L)])
                        acc_lo[c] = acc_lo[c] + lo
                        acc_hi[c] = acc_hi[c] + hi
                for c in range(nb):
                    acc_v[slot, si, pl.ds(c * 2 * _L, _L)] = acc_lo[c]
                    acc_v[slot, si, pl.ds(c * 2 * _L + _L, _L)] = acc_hi[c]
                return 0

            lax.fori_loop(0, CS, seg_body, 0)
            out_write(h, slot)
            return 0

        lax.fori_loop(0, nchunk, chunk_body, 0)
        out_done(nchunk - 2, lax.rem(nchunk - 2, 2))
        out_done(nchunk - 1, lax.rem(nchunk - 1, 2))

    return pl.kernel(
        body,
        out_type=jax.ShapeDtypeStruct((n_seg, D), jnp.float32),
        mesh=mesh,
        scratch_types=[
            pltpu.VMEM((segs_per_w * S,), jnp.int32),
            pltpu.VMEM((2, CS * S, D // 2), jnp.int32),
            pltpu.VMEM((2, CS, D), jnp.float32),
            pltpu.SemaphoreType.DMA((2,)),
            pltpu.SemaphoreType.DMA((2,)),
        ],
        compiler_params=pltpu.CompilerParams(needs_layout_passes=False,
                                             use_tc_tiling_on_sc=False),
    )


def _attention_body(sums_ref, q_ref):
    E = sums_ref.shape[2] // 4
    m = [sums_ref[:, :, pl.ds(t * E, E)] for t in range(4)]
    mA = (m[0], m[1], m[2])
    mC = (m[1], m[2], m[3])
    q = jnp.mean(mC[0], axis=1)
    for h in (1, 2):
        p = jnp.sum(mA[h] * q[:, None, :], axis=2)
        attn = jax.nn.softmax(p, axis=1)
        q = q + jnp.sum(attn[:, :, None] * mC[h], axis=1)
    q_ref[...] = q


def _attention(sums, E, interpret=False):
    B, M, D = sums.shape
    bb = 128
    return pl.pallas_call(
        _attention_body,
        grid=(B // bb,),
        in_specs=[pl.BlockSpec((bb, M, D), lambda i: (i, 0, 0))],
        out_specs=pl.BlockSpec((bb, E), lambda i: (i, 0)),
        out_shape=jax.ShapeDtypeStruct((B, E), jnp.float32),
        interpret=interpret,
    )(sums)


def kernel(context, A0, A1, A2, C_last):
    B, M, S = context.shape
    E = A0.shape[1]
    n_seg = B * M
    ctx = context.reshape(n_seg * S)
    def rnd(x):
        b = lax.bitcast_convert_type(x, jnp.int32)
        odd = lax.bitwise_and(lax.shift_right_logical(b, 16), jnp.int32(1))
        return lax.shift_right_logical(b + jnp.int32(0x7FFF) + odd, 16)

    eh = E // 2
    words = [lax.bitwise_or(rnd(t[:, :eh]),
                            lax.shift_left(rnd(t[:, eh:]), jnp.int32(16)))
             for t in (A0, A1, A2, C_last)]
    tab = jnp.concatenate(words, axis=1)
    seg_fn = _make_segment_sums(n_seg, S, 4 * E, CS=16)
    sums = seg_fn(ctx, tab)
    return _attention(sums.reshape(B, M, 4 * E), E)


# speedup vs baseline: 1.8225x; 1.8225x over previous
import jax
import jax.numpy as jnp
from jax import lax
from jax.experimental import pallas as pl
from jax.experimental.pallas import tpu as pltpu
from jax.experimental.pallas import tpu_sc as plsc

_NC, _NS, _L = 2, 16, 16
_NW = _NC * _NS


def _pack_body(tab_ref, out_ref):
    D = tab_ref.shape[1]
    E = D // 4
    eh = E // 2

    b = lax.bitcast_convert_type(tab_ref[...], jnp.int32)
    odd = lax.bitwise_and(lax.shift_right_logical(b, 16), jnp.int32(1))
    r = lax.shift_right_logical(b + jnp.int32(0x7FFF) + odd, 16)
    words = [lax.bitwise_or(
        r[:, t * E:t * E + eh],
        lax.shift_left(r[:, t * E + eh:(t + 1) * E], jnp.int32(16)))
        for t in range(4)]
    out_ref[...] = jnp.concatenate(words, axis=1)


def _make_pack(V, D, BV):
    return pl.pallas_call(
        _pack_body,
        grid=(V // BV,),
        in_specs=[pl.BlockSpec((BV, D), lambda i: (i, 0))],
        out_specs=pl.BlockSpec((BV, D // 2), lambda i: (i, 0)),
        out_shape=jax.ShapeDtypeStruct((V, D // 2), jnp.int32),
    )


def _make_segment_sums(n_seg, S, D, CS):
    mesh = plsc.VectorSubcoreMesh(
        core_axis_name="c", subcore_axis_name="s",
        num_cores=_NC, num_subcores=_NS)
    segs_per_w = n_seg // _NW
    nchunk = segs_per_w // CS
    nb = D // (2 * _L)

    def body(ctx_hbm, tab, out, idx_v, rows_v, acc_v, gsem, osem):
        wid = lax.axis_index("s") * _NC + lax.axis_index("c")
        seg0 = wid * segs_per_w

        def start_gather(h, slot):
            idx = idx_v.at[pl.ds(h * CS * S, CS * S)]
            pltpu.async_copy(tab.at[idx], rows_v.at[slot], gsem.at[slot])

        def gather_done(slot):
            idx = idx_v.at[pl.ds(0, CS * S)]
            pltpu.make_async_copy(tab.at[idx], rows_v.at[slot],
                                  gsem.at[slot]).wait()

        def out_write(h, slot):
            pltpu.async_copy(acc_v.at[slot], out.at[pl.ds(seg0 + h * CS, CS)],
                             osem.at[slot])

        def out_done(h, slot):
            pltpu.make_async_copy(acc_v.at[slot],
                                  out.at[pl.ds(seg0 + h * CS, CS)],
                                  osem.at[slot]).wait()

        pltpu.sync_copy(ctx_hbm.at[pl.ds(seg0 * S, segs_per_w * S)], idx_v)
        start_gather(0, 0)

        def chunk_body(h, _):
            slot = lax.rem(h, 2)

            @pl.when(h + 1 < nchunk)
            def _():
                start_gather(h + 1, 1 - slot)

            gather_done(slot)

            @pl.when(h >= 2)
            def _():
                out_done(h - 2, slot)

            def unpack(x):
                lo = plsc.bitcast(lax.shift_left(x, jnp.int32(16)),
                                  jnp.float32)
                hi = plsc.bitcast(lax.bitwise_and(x, jnp.int32(-65536)),
                                  jnp.float32)
                return lo, hi

            def seg_body(si, _):
                r = si * S
                acc_lo = [None] * nb
                acc_hi = [None] * nb
                for c in range(nb):
                    acc_lo[c], acc_hi[c] = unpack(
                        rows_v[slot, r, pl.ds(c * _L, _L)])
                for s in range(1, S):
                    for c in range(nb):
                        lo, hi = unpack(
                            rows_v[slot, r + s, pl.ds(c * _L, _L)])
                        acc_lo[c] = acc_lo[c] + lo
                        acc_hi[c] = acc_hi[c] + hi
                for c in range(nb):
                    acc_v[slot, si, pl.ds(c * 2 * _L, _L)] = acc_lo[c]
                    acc_v[slot, si, pl.ds(c * 2 * _L + _L, _L)] = acc_hi[c]
                return 0

            lax.fori_loop(0, CS, seg_body, 0)
            out_write(h, slot)
            return 0

        lax.fori_loop(0, nchunk, chunk_body, 0)
        out_done(nchunk - 2, lax.rem(nchunk - 2, 2))
        out_done(nchunk - 1, lax.rem(nchunk - 1, 2))

    return pl.kernel(
        body,
        out_type=jax.ShapeDtypeStruct((n_seg, D), jnp.float32),
        mesh=mesh,
        scratch_types=[
            pltpu.VMEM((segs_per_w * S,), jnp.int32),
            pltpu.VMEM((2, CS * S, D // 2), jnp.int32),
            pltpu.VMEM((2, CS, D), jnp.float32),
            pltpu.SemaphoreType.DMA((2,)),
            pltpu.SemaphoreType.DMA((2,)),
        ],
        compiler_params=pltpu.CompilerParams(needs_layout_passes=False,
                                             use_tc_tiling_on_sc=False),
    )


def _attention_body(sums_ref, q_ref):
    E = sums_ref.shape[2] // 4
    m = [sums_ref[:, :, pl.ds(t * E, E)] for t in range(4)]
    mA = (m[0], m[1], m[2])
    mC = (m[1], m[2], m[3])
    q = jnp.mean(mC[0], axis=1)
    for h in (1, 2):
        p = jnp.sum(mA[h] * q[:, None, :], axis=2)
        attn = jax.nn.softmax(p, axis=1)
        q = q + jnp.sum(attn[:, :, None] * mC[h], axis=1)
    q_ref[...] = q


def _attention(sums, E, interpret=False):
    B, M, D = sums.shape
    bb = 128
    return pl.pallas_call(
        _attention_body,
        grid=(B // bb,),
        in_specs=[pl.BlockSpec((bb, M, D), lambda i: (i, 0, 0))],
        out_specs=pl.BlockSpec((bb, E), lambda i: (i, 0)),
        out_shape=jax.ShapeDtypeStruct((B, E), jnp.float32),
        interpret=interpret,
    )(sums)


def kernel(context, A0, A1, A2, C_last):
    B, M, S = context.shape
    E = A0.shape[1]
    n_seg = B * M
    ctx = context.reshape(n_seg * S)
    V = A0.shape[0]
    tab_f = jnp.concatenate([A0, A1, A2, C_last], axis=1)
    tab = _make_pack(V, 4 * E, BV=2000)(tab_f)
    seg_fn = _make_segment_sums(n_seg, S, 4 * E, CS=16)
    sums = seg_fn(ctx, tab)
    return _attention(sums.reshape(B, M, 4 * E), E)


# speedup vs baseline: 2.0655x; 1.1333x over previous
import jax
import jax.numpy as jnp
from jax import lax
from jax.experimental import pallas as pl
from jax.experimental.pallas import tpu as pltpu
from jax.experimental.pallas import tpu_sc as plsc

_NC, _NS, _L = 2, 16, 16
_NW = _NC * _NS


def _make_segment_sums(n_seg, S, D, CS):
    mesh = plsc.VectorSubcoreMesh(
        core_axis_name="c", subcore_axis_name="s",
        num_cores=_NC, num_subcores=_NS)
    segs_per_w = n_seg // _NW
    nchunk = segs_per_w // CS
    nh = D // _L

    def body(ctx_hbm, tab, out, idx_v, rows_v, acc_v, gsem, osem):
        wid = lax.axis_index("s") * _NC + lax.axis_index("c")
        seg0 = wid * segs_per_w

        def start_gather(h, slot):
            idx = idx_v.at[pl.ds(h * CS * S, CS * S)]
            pltpu.async_copy(tab.at[idx], rows_v.at[slot], gsem.at[slot])

        def gather_done(slot):
            idx = idx_v.at[pl.ds(0, CS * S)]
            pltpu.make_async_copy(tab.at[idx], rows_v.at[slot],
                                  gsem.at[slot]).wait()

        def out_write(h, slot):
            pltpu.async_copy(acc_v.at[slot], out.at[pl.ds(seg0 + h * CS, CS)],
                             osem.at[slot])

        def out_done(h, slot):
            pltpu.make_async_copy(acc_v.at[slot],
                                  out.at[pl.ds(seg0 + h * CS, CS)],
                                  osem.at[slot]).wait()

        pltpu.sync_copy(ctx_hbm.at[pl.ds(seg0 * S, segs_per_w * S)], idx_v)
        start_gather(0, 0)

        def chunk_body(h, _):
            slot = lax.rem(h, 2)

            @pl.when(h + 1 < nchunk)
            def _():
                start_gather(h + 1, 1 - slot)

            gather_done(slot)

            @pl.when(h >= 2)
            def _():
                out_done(h - 2, slot)

            def seg_body(si, _):
                r = si * S
                accs = [rows_v[slot, r, pl.ds(c * _L, _L)] for c in range(nh)]
                for s in range(1, S):
                    for c in range(nh):
                        accs[c] = accs[c] + rows_v[slot, r + s,
                                                   pl.ds(c * _L, _L)]
                for c in range(nh):
                    acc_v[slot, si, pl.ds(c * _L, _L)] = accs[c]
                return 0

            lax.fori_loop(0, CS, seg_body, 0)
            out_write(h, slot)
            return 0

        lax.fori_loop(0, nchunk, chunk_body, 0)
        out_done(nchunk - 2, lax.rem(nchunk - 2, 2))
        out_done(nchunk - 1, lax.rem(nchunk - 1, 2))

    return pl.kernel(
        body,
        out_type=jax.ShapeDtypeStruct((n_seg, D), jnp.float32),
        mesh=mesh,
        scratch_types=[
            pltpu.VMEM((segs_per_w * S,), jnp.int32),
            pltpu.VMEM((2, CS * S, D), jnp.float32),
            pltpu.VMEM((2, CS, D), jnp.float32),
            pltpu.SemaphoreType.DMA((2,)),
            pltpu.SemaphoreType.DMA((2,)),
        ],
    )


def _attention_body(sums_ref, q_ref):
    E = sums_ref.shape[2] // 4
    m = [sums_ref[:, :, pl.ds(t * E, E)] for t in range(4)]
    mA = (m[0], m[1], m[2])
    mC = (m[1], m[2], m[3])
    q = jnp.mean(mC[0], axis=1)
    for h in (1, 2):
        p = jnp.sum(mA[h] * q[:, None, :], axis=2)
        attn = jax.nn.softmax(p, axis=1)
        q = q + jnp.sum(attn[:, :, None] * mC[h], axis=1)
    q_ref[...] = q


def _attention(sums, E, interpret=False):
    B, M, D = sums.shape
    bb = 128
    return pl.pallas_call(
        _attention_body,
        grid=(B // bb,),
        in_specs=[pl.BlockSpec((bb, M, D), lambda i: (i, 0, 0))],
        out_specs=pl.BlockSpec((bb, E), lambda i: (i, 0)),
        out_shape=jax.ShapeDtypeStruct((B, E), jnp.float32),
        interpret=interpret,
    )(sums)


def kernel(context, A0, A1, A2, C_last):
    B, M, S = context.shape
    E = A0.shape[1]
    tab = jnp.concatenate([A0, A1, A2, C_last], axis=1)
    B2 = B // 2
    seg_fn = _make_segment_sums(B2 * M, S, 4 * E, CS=16)
    qs = []
    for h in range(2):
        ctx_h = context[h * B2:(h + 1) * B2].reshape(B2 * M * S)
        sums_h = seg_fn(ctx_h, tab)
        qs.append(_attention(sums_h.reshape(B2, M, 4 * E), E))
    return jnp.concatenate(qs, axis=0)


# speedup vs baseline: 2.0802x; 1.0071x over previous
import jax
import jax.numpy as jnp
from jax import lax
from jax.experimental import pallas as pl
from jax.experimental.pallas import tpu as pltpu
from jax.experimental.pallas import tpu_sc as plsc

_NC, _NS, _L = 2, 16, 16
_NW = _NC * _NS


def _make_segment_sums(n_seg, S, D, CS):
    mesh = plsc.VectorSubcoreMesh(
        core_axis_name="c", subcore_axis_name="s",
        num_cores=_NC, num_subcores=_NS)
    segs_per_w = n_seg // _NW
    nchunk = segs_per_w // CS
    nh = D // _L

    def body(ctx_hbm, tab, out, idx_v, rows_v, acc_v, gsem, osem):
        wid = lax.axis_index("s") * _NC + lax.axis_index("c")
        seg0 = wid * segs_per_w

        def start_gather(h, slot):
            idx = idx_v.at[pl.ds(h * CS * S, CS * S)]
            pltpu.async_copy(tab.at[idx], rows_v.at[slot], gsem.at[slot])

        def gather_done(slot):
            idx = idx_v.at[pl.ds(0, CS * S)]
            pltpu.make_async_copy(tab.at[idx], rows_v.at[slot],
                                  gsem.at[slot]).wait()

        def out_write(h, slot):
            pltpu.async_copy(acc_v.at[slot], out.at[pl.ds(seg0 + h * CS, CS)],
                             osem.at[slot])

        def out_done(h, slot):
            pltpu.make_async_copy(acc_v.at[slot],
                                  out.at[pl.ds(seg0 + h * CS, CS)],
                                  osem.at[slot]).wait()

        pltpu.sync_copy(ctx_hbm.at[pl.ds(seg0 * S, segs_per_w * S)], idx_v)
        start_gather(0, 0)

        def chunk_body(h, _):
            slot = lax.rem(h, 2)

            @pl.when(h + 1 < nchunk)
            def _():
                start_gather(h + 1, 1 - slot)

            gather_done(slot)

            @pl.when(h >= 2)
            def _():
                out_done(h - 2, slot)

            def seg_body(si, _):
                r = si * S
                accs = [rows_v[slot, r, pl.ds(c * _L, _L)] for c in range(nh)]
                for s in range(1, S):
                    for c in range(nh):
                        accs[c] = accs[c] + rows_v[slot, r + s,
                                                   pl.ds(c * _L, _L)]
                for c in range(nh):
                    acc_v[slot, si, pl.ds(c * _L, _L)] = accs[c]
                return 0

            lax.fori_loop(0, CS, seg_body, 0)
            out_write(h, slot)
            return 0

        lax.fori_loop(0, nchunk, chunk_body, 0)
        out_done(nchunk - 2, lax.rem(nchunk - 2, 2))
        out_done(nchunk - 1, lax.rem(nchunk - 1, 2))

    return pl.kernel(
        body,
        out_type=jax.ShapeDtypeStruct((n_seg, D), jnp.float32),
        mesh=mesh,
        scratch_types=[
            pltpu.VMEM((segs_per_w * S,), jnp.int32),
            pltpu.VMEM((2, CS * S, D), jnp.float32),
            pltpu.VMEM((2, CS, D), jnp.float32),
            pltpu.SemaphoreType.DMA((2,)),
            pltpu.SemaphoreType.DMA((2,)),
        ],
    )


def _attention_body(sums_ref, q_ref):
    E = sums_ref.shape[2] // 4
    m = [sums_ref[:, :, pl.ds(t * E, E)] for t in range(4)]
    mA = (m[0], m[1], m[2])
    mC = (m[1], m[2], m[3])
    q = jnp.mean(mC[0], axis=1)
    for h in (1, 2):
        p = jnp.sum(mA[h] * q[:, None, :], axis=2)
        attn = jax.nn.softmax(p, axis=1)
        q = q + jnp.sum(attn[:, :, None] * mC[h], axis=1)
    q_ref[...] = q


def _attention(sums, E, interpret=False):
    B, M, D = sums.shape
    bb = 128
    return pl.pallas_call(
        _attention_body,
        grid=(B // bb,),
        in_specs=[pl.BlockSpec((bb, M, D), lambda i: (i, 0, 0))],
        out_specs=pl.BlockSpec((bb, E), lambda i: (i, 0)),
        out_shape=jax.ShapeDtypeStruct((B, E), jnp.float32),
        interpret=interpret,
    )(sums)


def kernel(context, A0, A1, A2, C_last):
    B, M, S = context.shape
    E = A0.shape[1]
    tab = jnp.concatenate([A0, A1, A2, C_last], axis=1)
    B2 = B // 4
    seg_fn = _make_segment_sums(B2 * M, S, 4 * E, CS=16)
    qs = []
    for h in range(4):
        ctx_h = context[h * B2:(h + 1) * B2].reshape(B2 * M * S)
        sums_h = seg_fn(ctx_h, tab)
        qs.append(_attention(sums_h.reshape(B2, M, 4 * E), E))
    return jnp.concatenate(qs, axis=0)
